# Initial kernel scaffold; baseline (speedup 1.0000x reference)
import jax
import jax.numpy as jnp
from jax.experimental import pallas as pl


def kernel(pred_rel_logits, pred_rel_vec, pred_rel_obj_logits, pred_rel_sub_logits, pred_rel_obj_box, pred_rel_sub_box, target_sizes):
    raise NotImplementedError("write your pallas kernel here")


# speedup vs baseline: 1.0114x; 1.0114x over previous
import functools

import jax
import jax.numpy as jnp
from jax.experimental import pallas as pl

_B = 2
_Q = 500
_REL_CLS = 51
_ENT_CLS = 150
_MME = 3
_NMAX = 300

_FLAT = _Q * _REL_CLS
_FLAT_PAD = 25600
_QPAD = 512
_ENT_PAD = 256
_JPAD = 384

_DN_T = (((0,), (0,)), ((), ()))
_DN_M = (((1,), (0,)), ((), ()))


def _kern(flat_ref, objl_ref, subl_ref, sbox_ref, obox_ref, scale_ref, out_ref):
    f32 = jnp.float32

    def fiota(shape, dim):
        return jax.lax.broadcasted_iota(jnp.int32, shape, dim).astype(f32)

    A = jax.nn.sigmoid(flat_ref[0])
    iota_flat = fiota((200, 128), 0) * 128.0 + fiota((200, 128), 1)
    lane512 = jax.lax.broadcasted_iota(jnp.int32, (1, _QPAD), 1)

    def s1(r, carry):
        a, accv, acci = carry
        m = jnp.max(a)
        j = jnp.min(jnp.where(a == m, iota_flat, 3.0e7))
        hit = lane512 == r
        accv = jnp.where(hit, m, accv)
        acci = jnp.where(hit, j, acci)
        a = jnp.where(iota_flat == j, -jnp.inf, a)
        return a, accv, acci

    zrow = jnp.zeros((1, _QPAD), f32)
    _, v, idx = jax.lax.fori_loop(0, _Q, s1, (A, zrow, zrow))
    q = jnp.floor(idx / float(_REL_CLS))
    rel_lab = idx - float(_REL_CLS) * q + 1.0

    lane256 = fiota((_QPAD, _ENT_PAD), 1)

    def top3(logits):
        d = jax.nn.sigmoid(logits)
        scr, lab = [], []
        for _ in range(_MME):
            m = jnp.max(d, axis=1, keepdims=True)
            l = jnp.min(jnp.where(d == m, lane256, 1.0e9), axis=1, keepdims=True)
            scr.append(m)
            lab.append(l)
            d = jnp.where(lane256 == l, -jnp.inf, d)
        return jnp.concatenate(scr, 1), jnp.concatenate(lab, 1)

    sub_scr, sub_lab = top3(subl_ref[0])
    obj_scr, obj_lab = top3(objl_ref[0])

    O2 = (fiota((_QPAD, _QPAD), 0) == q).astype(f32)
    dot = functools.partial(jax.lax.dot_general, preferred_element_type=f32,
                            precision=jax.lax.Precision.HIGHEST)
    sub_s3 = dot(sub_scr, O2, _DN_T)
    sub_l3 = dot(sub_lab, O2, _DN_T)
    obj_s3 = dot(obj_scr, O2, _DN_T)
    obj_l3 = dot(obj_lab, O2, _DN_T)

    sc = scale_ref[0]
    W, H = sc[0:1], sc[1:2]

    def to_xyxy_scaled(boxT):
        cx, cy, w, h = boxT[0:1], boxT[1:2], boxT[2:3], boxT[3:4]
        return jnp.concatenate(
            [(cx - 0.5 * w) * W, (cy - 0.5 * h) * H,
             (cx + 0.5 * w) * W, (cy + 0.5 * h) * H], 0)

    def second_conv(bT):
        r0, r1, r2, r3 = bT[0:1], bT[1:2], bT[2:3], bT[3:4]
        return (r0 - 0.5 * r2, r1 - 0.5 * r3, r0 + 0.5 * r2, r1 + 0.5 * r3)

    sbb = dot(to_xyxy_scaled(sbox_ref[0]), O2, _DN_M)
    obb = dot(to_xyxy_scaled(obox_ref[0]), O2, _DN_M)
    sx1, sy1, sx2, sy2 = second_conv(sbb)
    ox1, oy1, ox2, oy2 = second_conv(obb)
    area1 = (sx2 - sx1) * (sy2 - sy1)
    area2 = (ox2 - ox1) * (oy2 - oy1)
    iw = jnp.clip(jnp.minimum(sx2, ox2) - jnp.maximum(sx1, ox1), 0.0)
    ih = jnp.clip(jnp.minimum(sy2, oy2) - jnp.maximum(sy1, oy1), 0.0)
    inter = iw * ih
    union = area1 + area2 - inter
    iou = inter / (union + 1e-9)
    cw = jnp.clip(jnp.maximum(sx2, ox2) - jnp.minimum(sx1, ox1), 0.0)
    ch = jnp.clip(jnp.maximum(sy2, oy2) - jnp.minimum(sy1, oy1), 0.0)
    areac = cw * ch
    giou = iou - (areac - union) / (areac + 1e-9)

    scores3 = (v * sub_s3) * obj_s3
    valid = (giou < 0.95) & (sub_l3 != obj_l3) & (lane512 < _Q)
    iota3 = fiota((_MME, _QPAD), 1) * 3.0 + fiota((_MME, _QPAD), 0)
    M = jnp.where(valid, scores3, -jnp.inf)
    lane384 = jax.lax.broadcasted_iota(jnp.int32, (1, _JPAD), 1)

    def s4(r, carry):
        m_arr, accj = carry
        mx = jnp.max(m_arr)
        j = jnp.min(jnp.where(m_arr == mx, iota3, 1.0e9))
        accj = jnp.where(lane384 == r, j, accj)
        m_arr = jnp.where(iota3 == j, -jnp.inf, m_arr)
        return m_arr, accj

    _, J = jax.lax.fori_loop(0, _NMAX, s4, (M, jnp.zeros((1, _JPAD), f32)))
    I = jnp.floor(J / 3.0)
    K = J - 3.0 * I

    OI2 = (fiota((_QPAD, _JPAD), 0) == I).astype(f32)
    lab_g = dot(rel_lab, OI2, _DN_M)
    v_g = dot(v, OI2, _DN_M)
    q_g = dot(q, OI2, _DN_M)
    s_cols = dot(scores3, OI2, _DN_M)
    ksel = fiota((_MME, _JPAD), 0) == K
    s_g = jnp.sum(jnp.where(ksel, s_cols, 0.0), axis=0, keepdims=True)
    out_ref[0] = jnp.concatenate([J, J + 1500.0, lab_g, s_g, v_g, q_g], 0)


@jax.jit
def kernel(pred_rel_logits, pred_rel_vec, pred_rel_obj_logits,
           pred_rel_sub_logits, pred_rel_obj_box, pred_rel_sub_box,
           target_sizes):
    del pred_rel_vec
    img_h, img_w = target_sizes[:, 0], target_sizes[:, 1]
    scale = jnp.stack([img_w, img_h, img_w, img_h], axis=1)[:, :, None]

    neg = jnp.float32(-1e30)
    flat = pred_rel_logits.reshape(_B, _FLAT)
    flat = jnp.pad(flat, ((0, 0), (0, _FLAT_PAD - _FLAT)), constant_values=neg)
    flat = flat.reshape(_B, 200, 128)

    def pad_ent(x):
        return jnp.pad(x, ((0, 0), (0, _QPAD - _Q), (0, _ENT_PAD - _ENT_CLS)),
                       constant_values=neg)

    objl = pad_ent(pred_rel_obj_logits)
    subl = pad_ent(pred_rel_sub_logits)

    def pad_box(x):
        return jnp.pad(x.transpose(0, 2, 1), ((0, 0), (0, 0), (0, _QPAD - _Q)))

    sbox = pad_box(pred_rel_sub_box)
    obox = pad_box(pred_rel_obj_box)

    spec = lambda *s: pl.BlockSpec((1,) + s, lambda b: (b,) + (0,) * len(s))
    out = pl.pallas_call(
        _kern,
        grid=(_B,),
        in_specs=[spec(200, 128), spec(_QPAD, _ENT_PAD), spec(_QPAD, _ENT_PAD),
                  spec(4, _QPAD), spec(4, _QPAD), spec(4, 1)],
        out_specs=spec(6, _JPAD),
        out_shape=jax.ShapeDtypeStruct((_B, 6, _JPAD), jnp.float32),
    )(flat, objl, subl, sbox, obox, scale)

    return tuple(out[b, :, :_NMAX].T for b in range(_B))


# speedup vs baseline: 2.2986x; 2.2728x over previous
import functools

import jax
import jax.numpy as jnp
from jax.experimental import pallas as pl

_B = 2
_Q = 500
_REL_CLS = 51
_ENT_CLS = 150
_MME = 3
_NMAX = 300

_FLAT = _Q * _REL_CLS
_FLAT_PAD = 25600
_QPAD = 512
_ENT_PAD = 256
_JPAD = 384

_DN_T = (((1,), (1,)), ((0,), (0,)))
_DN_M = (((2,), (1,)), ((0,), (0,)))


def _kern(flat_ref, objl_ref, subl_ref, sbox_ref, obox_ref, scale_ref, out_ref):
    f32 = jnp.float32

    def fiota(shape, dim):
        return jax.lax.broadcasted_iota(jnp.int32, shape, dim).astype(f32)

    A = jax.nn.sigmoid(flat_ref[...])
    iota_flat = fiota((1, 200, 128), 1) * 128.0 + fiota((1, 200, 128), 2)
    lane512 = jax.lax.broadcasted_iota(jnp.int32, (1, 1, _QPAD), 2)

    def s1(r, carry):
        a, accv, acci = carry
        m = jnp.max(a, axis=(1, 2), keepdims=True)
        j = jnp.min(jnp.where(a == m, iota_flat, 3.0e7),
                    axis=(1, 2), keepdims=True)
        hit = lane512 == r
        accv = jnp.where(hit, m, accv)
        acci = jnp.where(hit, j, acci)
        a = jnp.where(iota_flat == j, -jnp.inf, a)
        return a, accv, acci

    zrow = jnp.zeros((_B, 1, _QPAD), f32)
    _, v, idx = jax.lax.fori_loop(0, _Q, s1, (A, zrow, zrow))
    q = jnp.floor(idx / float(_REL_CLS))
    rel_lab = idx - float(_REL_CLS) * q + 1.0

    lane256 = fiota((1, _QPAD, _ENT_PAD), 2)

    def top3(logits):
        d = jax.nn.sigmoid(logits)
        scr, lab = [], []
        for _ in range(_MME):
            m = jnp.max(d, axis=2, keepdims=True)
            l = jnp.min(jnp.where(d == m, lane256, 1.0e9), axis=2, keepdims=True)
            scr.append(m)
            lab.append(l)
            d = jnp.where(lane256 == l, -jnp.inf, d)
        return jnp.concatenate(scr, 2), jnp.concatenate(lab, 2)

    sub_scr, sub_lab = top3(subl_ref[...])
    obj_scr, obj_lab = top3(objl_ref[...])

    O2 = (fiota((1, _QPAD, 1), 1) == q).astype(f32)
    dot = functools.partial(jax.lax.dot_general, preferred_element_type=f32,
                            precision=jax.lax.Precision.HIGHEST)
    sub_s3 = dot(sub_scr, O2, _DN_T)
    sub_l3 = dot(sub_lab, O2, _DN_T)
    obj_s3 = dot(obj_scr, O2, _DN_T)
    obj_l3 = dot(obj_lab, O2, _DN_T)

    sc = scale_ref[...]
    W, H = sc[:, 0:1], sc[:, 1:2]

    def to_xyxy_scaled(boxT):
        cx, cy = boxT[:, 0:1], boxT[:, 1:2]
        w, h = boxT[:, 2:3], boxT[:, 3:4]
        return jnp.concatenate(
            [(cx - 0.5 * w) * W, (cy - 0.5 * h) * H,
             (cx + 0.5 * w) * W, (cy + 0.5 * h) * H], 1)

    def second_conv(bT):
        r0, r1, r2, r3 = bT[:, 0:1], bT[:, 1:2], bT[:, 2:3], bT[:, 3:4]
        return (r0 - 0.5 * r2, r1 - 0.5 * r3, r0 + 0.5 * r2, r1 + 0.5 * r3)

    sbb = dot(to_xyxy_scaled(sbox_ref[...]), O2, _DN_M)
    obb = dot(to_xyxy_scaled(obox_ref[...]), O2, _DN_M)
    sx1, sy1, sx2, sy2 = second_conv(sbb)
    ox1, oy1, ox2, oy2 = second_conv(obb)
    area1 = (sx2 - sx1) * (sy2 - sy1)
    area2 = (ox2 - ox1) * (oy2 - oy1)
    iw = jnp.clip(jnp.minimum(sx2, ox2) - jnp.maximum(sx1, ox1), 0.0)
    ih = jnp.clip(jnp.minimum(sy2, oy2) - jnp.maximum(sy1, oy1), 0.0)
    inter = iw * ih
    union = area1 + area2 - inter
    iou = inter / (union + 1e-9)
    cw = jnp.clip(jnp.maximum(sx2, ox2) - jnp.minimum(sx1, ox1), 0.0)
    ch = jnp.clip(jnp.maximum(sy2, oy2) - jnp.minimum(sy1, oy1), 0.0)
    areac = cw * ch
    giou = iou - (areac - union) / (areac + 1e-9)

    scores3 = (v * sub_s3) * obj_s3
    valid = (giou < 0.95) & (sub_l3 != obj_l3) & (lane512 < _Q)
    iota3 = fiota((1, _MME, _QPAD), 2) * 3.0 + fiota((1, _MME, _QPAD), 1)
    M = jnp.where(valid, scores3, -jnp.inf)
    lane384 = jax.lax.broadcasted_iota(jnp.int32, (1, 1, _JPAD), 2)

    def s4(r, carry):
        m_arr, accj = carry
        mx = jnp.max(m_arr, axis=(1, 2), keepdims=True)
        j = jnp.min(jnp.where(m_arr == mx, iota3, 1.0e9),
                    axis=(1, 2), keepdims=True)
        accj = jnp.where(lane384 == r, j, accj)
        m_arr = jnp.where(iota3 == j, -jnp.inf, m_arr)
        return m_arr, accj

    _, J = jax.lax.fori_loop(0, _NMAX, s4,
                             (M, jnp.zeros((_B, 1, _JPAD), f32)))
    I = jnp.floor(J / 3.0)
    K = J - 3.0 * I

    OI2 = (fiota((1, _QPAD, 1), 1) == I).astype(f32)
    lab_g = dot(rel_lab, OI2, _DN_M)
    v_g = dot(v, OI2, _DN_M)
    q_g = dot(q, OI2, _DN_M)
    s_cols = dot(scores3, OI2, _DN_M)
    ksel = fiota((1, _MME, 1), 1) == K
    s_g = jnp.sum(jnp.where(ksel, s_cols, 0.0), axis=1, keepdims=True)
    out_ref[...] = jnp.concatenate([J, J + 1500.0, lab_g, s_g, v_g, q_g], 1)


@jax.jit
def kernel(pred_rel_logits, pred_rel_vec, pred_rel_obj_logits,
           pred_rel_sub_logits, pred_rel_obj_box, pred_rel_sub_box,
           target_sizes):
    del pred_rel_vec
    img_h, img_w = target_sizes[:, 0], target_sizes[:, 1]
    scale = jnp.stack([img_w, img_h, img_w, img_h], axis=1)[:, :, None]

    neg = jnp.float32(-1e30)
    flat = pred_rel_logits.reshape(_B, _FLAT)
    flat = jnp.pad(flat, ((0, 0), (0, _FLAT_PAD - _FLAT)), constant_values=neg)
    flat = flat.reshape(_B, 200, 128)

    def pad_ent(x):
        return jnp.pad(x, ((0, 0), (0, _QPAD - _Q), (0, _ENT_PAD - _ENT_CLS)),
                       constant_values=neg)

    objl = pad_ent(pred_rel_obj_logits)
    subl = pad_ent(pred_rel_sub_logits)

    def pad_box(x):
        return jnp.pad(x.transpose(0, 2, 1), ((0, 0), (0, 0), (0, _QPAD - _Q)))

    sbox = pad_box(pred_rel_sub_box)
    obox = pad_box(pred_rel_obj_box)

    out = pl.pallas_call(
        _kern,
        out_shape=jax.ShapeDtypeStruct((_B, 6, _JPAD), jnp.float32),
    )(flat, objl, subl, sbox, obox, scale)

    return tuple(out[b, :, :_NMAX].T for b in range(_B))
